# Initial kernel scaffold; baseline (speedup 1.0000x reference)
import jax
import jax.numpy as jnp
from jax.experimental import pallas as pl


def kernel(x, edge_index, Wc, bc, Ws, bs, Wl0, bl0, Wl1, bl1, Wl2, bl2, Wl3, bl3, Wl4, bl4, Wl5, bl5):
    raise NotImplementedError("write your pallas kernel here")


# speedup vs baseline: 1.0591x; 1.0591x over previous
import functools

import jax
import jax.numpy as jnp
from jax import lax
from jax.experimental import pallas as pl

N = 10000
E = 160000
D = 256
NUM_CONVS = 5
KS = [8000, 6400, 5120, 4096, 3277]
MB = 400


def _mm_kernel(a_ref, w_ref, b_ref, o_ref, *, relu):
    acc = lax.dot_general(a_ref[...], w_ref[...], (((1,), (0,)), ((), ())),
                          precision="default", preferred_element_type=jnp.float32)
    acc = acc + b_ref[...]
    if relu:
        acc = jnp.maximum(acc, 0.0)
    o_ref[...] = acc


def _matmul(a, w, b, relu):
    m, k = a.shape
    n = w.shape[1]
    grid = (m // MB,)
    return pl.pallas_call(
        functools.partial(_mm_kernel, relu=relu),
        grid=grid,
        in_specs=[
            pl.BlockSpec((MB, k), lambda i: (i, 0)),
            pl.BlockSpec((k, n), lambda i: (0, 0)),
            pl.BlockSpec((1, n), lambda i: (0, 0)),
        ],
        out_specs=pl.BlockSpec((MB, n), lambda i: (i, 0)),
        out_shape=jax.ShapeDtypeStruct((m, n), jnp.float32),
    )(a, w, b)


def _readout_kernel(f_ref, m_ref, o_ref):
    i = pl.program_id(0)
    f = f_ref[...]
    msk = m_ref[...]
    bsum = jnp.sum(f, axis=0, keepdims=True)
    bmax = jnp.max(jnp.where(msk > 0, f, -1e30), axis=0, keepdims=True)

    @pl.when(i == 0)
    def _():
        o_ref[:, :D] = bsum
        o_ref[:, D:] = bmax

    @pl.when(i > 0)
    def _():
        o_ref[:, :D] = o_ref[:, :D] + bsum
        o_ref[:, D:] = jnp.maximum(o_ref[:, D:], bmax)


def _readout(feat, mask2d):
    return pl.pallas_call(
        _readout_kernel,
        grid=(N // MB,),
        in_specs=[
            pl.BlockSpec((MB, D), lambda i: (i, 0)),
            pl.BlockSpec((MB, D), lambda i: (i, 0)),
        ],
        out_specs=pl.BlockSpec((1, 2 * D), lambda i: (0, 0)),
        out_shape=jax.ShapeDtypeStruct((1, 2 * D), jnp.float32),
    )(feat, mask2d)


def _mlp_layer_kernel(a_ref, w_ref, b_ref, o_ref):
    acc = lax.dot_general(a_ref[...], w_ref[...], (((1,), (0,)), ((), ())),
                          precision="default", preferred_element_type=jnp.float32)
    acc = acc + b_ref[...]
    o_ref[...] = jnp.clip(acc, -10.0, 10.0)


def _mlp_layer(a, w, b):
    k = a.shape[1]
    n = w.shape[1]
    del k
    return pl.pallas_call(
        _mlp_layer_kernel,
        out_shape=jax.ShapeDtypeStruct((8, n), jnp.float32),
    )(a, w, b)


def _pad_cols(x, n):
    return jnp.pad(x, ((0, 0), (0, n - x.shape[1])))


def kernel(x, edge_index, Wc, bc, Ws, bs, Wl0, bl0, Wl1, bl1, Wl2, bl2,
           Wl3, bl3, Wl4, bl4, Wl5, bl5):
    src = edge_index[0]
    dst = edge_index[1]
    node_mask = jnp.ones((N,), dtype=x.dtype)
    feat = x
    readouts = []
    perms = []
    for i in range(NUM_CONVS):
        emask = node_mask[src] * node_mask[dst]
        deg_out = jax.ops.segment_sum(emask, src, num_segments=N)
        deg_in = jax.ops.segment_sum(emask, dst, num_segments=N)
        norm_out = 1.0 / jnp.sqrt(jnp.maximum(deg_out, 1.0))
        norm_in = 1.0 / jnp.sqrt(jnp.maximum(deg_in, 1.0))

        hpre = feat * norm_out[:, None]
        msg = hpre[src] * emask[:, None]
        agg = jax.ops.segment_sum(msg, dst, num_segments=N)
        agg = agg * norm_in[:, None]
        h = _matmul(agg, Wc[i], bc[i][None, :], relu=True)

        hs = h * norm_out[:, None]
        msg2 = hs[src] * emask[:, None]
        agg2 = jax.ops.segment_sum(msg2, dst, num_segments=N)
        agg2 = agg2 * norm_in[:, None]
        score = _matmul(agg2, _pad_cols(Ws[i], 128), _pad_cols(bs[i][None, :], 128),
                        relu=False)[:, 0]

        masked_score = jnp.where(node_mask > 0, score, -1e30)
        _, perm = lax.top_k(masked_score, KS[i])
        new_mask = jnp.zeros((N,), dtype=x.dtype).at[perm].set(1.0)
        feat = h * jnp.tanh(score)[:, None] * new_mask[:, None]
        node_mask = new_mask

        ro = _readout(feat, jnp.broadcast_to(new_mask[:, None], (N, D)))
        readouts.append(jnp.concatenate([ro[:, :D] / KS[i], ro[:, D:]], axis=-1))
        perms.append(perm)

    fr = jnp.concatenate(readouts, axis=-1)
    fr = jnp.clip(fr, -10.0, 10.0)
    fr = jnp.pad(fr, ((0, 7), (0, 128 - fr.shape[1] % 128 if fr.shape[1] % 128 else 0)))
    dims_pad = [2688, 1792, 1152, 768, 512, 128]
    layers = [(Wl0, bl0), (Wl1, bl1), (Wl2, bl2), (Wl3, bl3), (Wl4, bl4), (Wl5, bl5)]
    a = fr
    for (w, b), np_ in zip(layers, dims_pad):
        wp = jnp.pad(w, ((0, a.shape[1] - w.shape[0]), (0, np_ - w.shape[1])))
        bp = _pad_cols(b[None, :], np_)
        a = _mlp_layer(a, wp, bp)
    val = jnp.clip(a[0:1, 0], -10.0, 10.0) * 100.0
    return val, jnp.concatenate(perms).astype(jnp.float32)
